# baseline (speedup 1.0000x reference)
import functools

import jax
import jax.numpy as jnp
from jax import lax
from jax.experimental import pallas as pl
from jax.experimental.pallas import tpu as pltpu
from jax.experimental.pallas import tpu_sc as plsc

_K = 128
_LANES = 16


@functools.partial(jax.jit, static_argnames=("nc", "ns"))
def _transpose_table(wt, t_tail, *, nc, ns):
    d, v = wt.shape
    nw = nc * ns
    n_full = v // _K
    v_tail = v - n_full * _K
    n_iter = (n_full + nw - 1) // nw
    n_iter = n_iter + (n_iter % 2)

    mesh = plsc.VectorSubcoreMesh(core_axis_name="c", subcore_axis_name="s")

    @functools.partial(
        pl.kernel,
        out_type=jax.ShapeDtypeStruct((v, d), jnp.float32),
        mesh=mesh,
        scratch_types=[
            pltpu.VMEM((d, _K), jnp.float32),
            pltpu.VMEM((d, _K), jnp.float32),
            pltpu.VMEM((_K, d), jnp.float32),
            pltpu.VMEM((_K, d), jnp.float32),
            pltpu.SemaphoreType.DMA,
            pltpu.SemaphoreType.DMA,
            pltpu.SemaphoreType.DMA,
            pltpu.SemaphoreType.DMA,
        ],
        compiler_params=pltpu.CompilerParams(needs_layout_passes=False),
    )
    def tr_kernel(
        wt_hbm, tail_hbm, out_hbm, in0, in1, asm0, asm1, si0, si1, so0, so1
    ):
        wid = lax.axis_index("s") * nc + lax.axis_index("c")
        ins = ((in0, si0), (in1, si1))
        asms = ((asm0, so0), (asm1, so1))
        lane = lax.iota(jnp.int32, _LANES)
        rvecs = [lane + l * _LANES for l in range(_K // _LANES)]

        def fire(it, h):
            buf, sem = ins[h]
            vc = wid + it * nw

            @pl.when(vc < n_full)
            def _():
                pltpu.async_copy(wt_hbm.at[:, pl.ds(vc * _K, _K)], buf, sem)

        def drain_prior_store(it, h):
            asm, osem = asms[h]
            vcp = wid + (it - 2) * nw

            @pl.when(jnp.logical_and(vcp >= 0, vcp < n_full))
            def _():
                pltpu.make_async_copy(
                    out_hbm.at[pl.ds(0, _K)], asm, osem
                ).wait()

        def scatter_block(buf, asm, n_lane_groups):
            for dr in range(d):
                cvec = jnp.zeros_like(lane) + dr
                for l in range(n_lane_groups):
                    val = buf[dr, pl.ds(l * _LANES, _LANES)]
                    plsc.store_scatter(asm, [rvecs[l], cvec], val)

        def assemble(it, h):
            buf, sem = ins[h]
            asm, osem = asms[h]
            vc = wid + it * nw
            drain_prior_store(it, h)

            @pl.when(vc < n_full)
            def _():
                pltpu.make_async_copy(
                    wt_hbm.at[:, pl.ds(0, _K)], buf, sem
                ).wait()
                scatter_block(buf, asm, _K // _LANES)
                pltpu.async_copy(asm, out_hbm.at[pl.ds(vc * _K, _K)], osem)

        fire(0, 0)
        fire(1, 1)

        @pl.loop(0, n_iter - 2, step=2)
        def _grp(i):
            for h in range(2):
                it = i + h
                assemble(it, h)
                fire(it + 2, h)

        for it in (n_iter - 2, n_iter - 1):
            assemble(it, it % 2)
        for it in (n_iter, n_iter + 1):
            drain_prior_store(it, it % 2)

        @pl.when(wid == 0)
        def _():
            pltpu.sync_copy(tail_hbm, asm0.at[pl.ds(0, v_tail)])
            pltpu.sync_copy(
                asm0.at[pl.ds(0, v_tail)],
                out_hbm.at[pl.ds(n_full * _K, v_tail)],
            )

    return tr_kernel(wt, t_tail)


@functools.partial(jax.jit, static_argnames=("nc", "ns"))
def _emb_gather(idx, table, *, nc, ns):
    nw = nc * ns
    _, n_chunks, k = idx.shape
    _, d = table.shape
    b = nw * n_chunks * k
    b_per_w = n_chunks * k

    mesh = plsc.VectorSubcoreMesh(core_axis_name="c", subcore_axis_name="s")

    @functools.partial(
        pl.kernel,
        out_type=jax.ShapeDtypeStruct((b, d), jnp.float32),
        mesh=mesh,
        scratch_types=[
            pltpu.VMEM((n_chunks, k), jnp.int32),
            pltpu.VMEM((k, d), jnp.float32),
            pltpu.VMEM((k, d), jnp.float32),
            pltpu.SemaphoreType.DMA,
            pltpu.SemaphoreType.DMA,
        ],
        compiler_params=pltpu.CompilerParams(use_tc_tiling_on_sc=True),
    )
    def emb_kernel(idx_hbm, table_hbm, out_hbm, idx_v, rows0, rows1, sem0, sem1):
        wid = lax.axis_index("s") * nc + lax.axis_index("c")
        base = wid * b_per_w
        pltpu.sync_copy(idx_hbm.at[wid], idx_v)

        halves = ((rows0, sem0), (rows1, sem1))

        def fire(gi, h):
            rows, sem = halves[h]
            for j16 in range(k // _LANES):
                vvec = idx_v[gi, pl.ds(j16 * _LANES, _LANES)]
                for j in range(_LANES):
                    r = j16 * _LANES + j
                    pltpu.async_copy(
                        table_hbm.at[pl.ds(vvec[j], 1)],
                        rows.at[pl.ds(r, 1)],
                        sem,
                    )

        def drain_store(gi, h):
            rows, sem = halves[h]
            pltpu.make_async_copy(
                table_hbm.at[pl.ds(0, k)], rows, sem
            ).wait()
            pltpu.sync_copy(rows, out_hbm.at[pl.ds(base + gi * k, k)])

        fire(0, 0)
        fire(1, 1)

        @pl.loop(0, n_chunks - 2, step=2)
        def _grp(i):
            for h in range(2):
                gi = i + h
                drain_store(gi, h)
                fire(gi + 2, h)

        for gi in (n_chunks - 2, n_chunks - 1):
            drain_store(gi, gi % 2)

    return emb_kernel(idx, table)


def kernel(inputs, weight):
    b0, s = inputs.shape
    v, d = weight.shape
    b = b0 * s
    info = plsc.get_sparse_core_info()
    nc, ns = info.num_cores, info.num_subcores
    nw = nc * ns
    idx = inputs.reshape(nw, b // (nw * _K), _K).astype(jnp.int32)
    n_full = v // _K
    table = _transpose_table(weight.T, weight[n_full * _K :], nc=nc, ns=ns)
    out = _emb_gather(idx, table, nc=nc, ns=ns)
    return out.reshape(b0, s, d)
